# baseline (speedup 1.0000x reference)
import jax
import jax.numpy as jnp
from jax.experimental import pallas as pl
from jax.experimental.pallas import tpu as pltpu

_GRID = 50
_XB = 10000 // _GRID
_EB = 320000 // _GRID


def _copy_body(x_ref, e_ref, xo_ref, eo_ref):
    xo_ref[...] = x_ref[...]
    eo_ref[...] = e_ref[...]


def kernel(x, edge_index, edge_attr):
    del edge_index
    x_out, e_out = pl.pallas_call(
        _copy_body,
        grid=(_GRID,),
        in_specs=[
            pl.BlockSpec((_XB, 128), lambda i: (i, jnp.int32(0))),
            pl.BlockSpec((_EB, 16), lambda i: (i, jnp.int32(0))),
        ],
        out_specs=[
            pl.BlockSpec((_XB, 128), lambda i: (i, jnp.int32(0))),
            pl.BlockSpec((_EB, 16), lambda i: (i, jnp.int32(0))),
        ],
        out_shape=[
            jax.ShapeDtypeStruct(x.shape, x.dtype),
            jax.ShapeDtypeStruct(edge_attr.shape, edge_attr.dtype),
        ],
        compiler_params=pltpu.CompilerParams(
            dimension_semantics=("arbitrary",),
        ),
    )(x, edge_attr)
    return (x_out, e_out)


# speedup vs baseline: 1.0064x; 1.0064x over previous
import jax
import jax.numpy as jnp
from jax.experimental import pallas as pl
from jax.experimental.pallas import tpu as pltpu

_GRID = 25
_XB = 10000 // _GRID
_EB = 320000 // _GRID


def _copy_body(x_ref, e_ref, xo_ref, eo_ref):
    xo_ref[...] = x_ref[...]
    eo_ref[...] = e_ref[...]


def kernel(x, edge_index, edge_attr):
    del edge_index
    x_out, e_out = pl.pallas_call(
        _copy_body,
        grid=(_GRID,),
        in_specs=[
            pl.BlockSpec((_XB, 128), lambda i: (i, jnp.int32(0))),
            pl.BlockSpec((_EB, 16), lambda i: (i, jnp.int32(0))),
        ],
        out_specs=[
            pl.BlockSpec((_XB, 128), lambda i: (i, jnp.int32(0))),
            pl.BlockSpec((_EB, 16), lambda i: (i, jnp.int32(0))),
        ],
        out_shape=[
            jax.ShapeDtypeStruct(x.shape, x.dtype),
            jax.ShapeDtypeStruct(edge_attr.shape, edge_attr.dtype),
        ],
        compiler_params=pltpu.CompilerParams(
            dimension_semantics=("arbitrary",),
        ),
    )(x, edge_attr)
    return (x_out, e_out)
